# baseline (speedup 1.0000x reference)
import functools

import jax
import jax.numpy as jnp
from jax import lax
from jax.experimental import pallas as pl
from jax.experimental.pallas import tpu as pltpu
from jax.experimental.pallas import tpu_sc as plsc

NC = 2
NS = 16
NW = NC * NS
CHUNK = 128


def _sum_parts(ref):
    tot = ref[0]
    for k in range(1, ref.shape[0]):
        tot = tot + ref[k]
    return tot


def _mm_body(out_dtype, x_ref, w_ref, b_ref, o_ref):
    o_ref[...] = (
        jnp.dot(x_ref[...], w_ref[...], preferred_element_type=jnp.float32)
        + b_ref[...]
    ).astype(out_dtype)


def _combine_mm_body(out_dtype, s_ref, acc_ref, deg_ref, w_ref, b_ref, o_ref):
    deg = _sum_parts(deg_ref)[:, 0:1]
    rdeg = 1.0 / jnp.maximum(deg, 1.0)
    acc = _sum_parts(acc_ref).astype(jnp.float32)
    h = jnp.maximum(s_ref[...] + acc * rdeg, 0.0)
    o_ref[...] = (
        jnp.dot(h, w_ref[...], preferred_element_type=jnp.float32) + b_ref[...]
    ).astype(out_dtype)


def _combine_body(s_ref, acc_ref, deg_ref, o_ref):
    deg = _sum_parts(deg_ref)[:, 0:1]
    rdeg = 1.0 / jnp.maximum(deg, 1.0)
    acc = _sum_parts(acc_ref).astype(jnp.float32)
    o_ref[...] = jnp.maximum(s_ref[...] + acc * rdeg, 0.0)


def _mm(x, w, b, out_dtype, blk):
    n, d = x.shape
    grid = n // blk
    return pl.pallas_call(
        functools.partial(_mm_body, out_dtype),
        grid=(grid,),
        in_specs=[
            pl.BlockSpec((blk, d), lambda i: (i, 0)),
            pl.BlockSpec((d, d), lambda i: (0, 0)),
            pl.BlockSpec((1, d), lambda i: (0, 0)),
        ],
        out_specs=pl.BlockSpec((blk, d), lambda i: (i, 0)),
        out_shape=jax.ShapeDtypeStruct((n, d), out_dtype),
    )(x, w, b.reshape(1, d))


def _combine_mm(s, acc, deg, w, b, out_dtype, blk):
    n, d = s.shape
    grid = n // blk
    na = acc.shape[0]
    return pl.pallas_call(
        functools.partial(_combine_mm_body, out_dtype),
        grid=(grid,),
        in_specs=[
            pl.BlockSpec((blk, d), lambda i: (i, 0)),
            pl.BlockSpec((na, blk, d), lambda i: (0, i, 0)),
            pl.BlockSpec((na, blk, 8), lambda i: (0, i, 0)),
            pl.BlockSpec((d, d), lambda i: (0, 0)),
            pl.BlockSpec((1, d), lambda i: (0, 0)),
        ],
        out_specs=pl.BlockSpec((blk, d), lambda i: (i, 0)),
        out_shape=jax.ShapeDtypeStruct((n, d), out_dtype),
    )(s, acc, deg, w, b.reshape(1, d))


def _combine(s, acc, deg, n_out, blk):
    d = s.shape[1]
    grid = n_out // blk
    na = acc.shape[0]
    return pl.pallas_call(
        _combine_body,
        grid=(grid,),
        in_specs=[
            pl.BlockSpec((blk, d), lambda i: (i, 0)),
            pl.BlockSpec((na, blk, d), lambda i: (0, i, 0)),
            pl.BlockSpec((na, blk, 8), lambda i: (0, i, 0)),
        ],
        out_specs=pl.BlockSpec((blk, d), lambda i: (i, 0)),
        out_shape=jax.ShapeDtypeStruct((n_out, d), jnp.float32),
    )(s, acc, deg)


def _make_edge_pass(np_, d, nch0, nch1, with_deg):
    rows_per_tile = np_ // NS
    grp = 8
    assert nch0 % grp == 0 and nch1 % grp == 0
    nacc = NC
    mesh = plsc.VectorSubcoreMesh(core_axis_name="c", subcore_axis_name="s")

    out_type = [jax.ShapeDtypeStruct((nacc, np_, d), jnp.bfloat16)]
    scratch = [
        pltpu.VMEM((grp, CHUNK), jnp.int32),
        pltpu.VMEM((grp, CHUNK), jnp.int32),
        pltpu.VMEM((CHUNK, d), jnp.bfloat16),
        pltpu.VMEM((CHUNK, d), jnp.bfloat16),
        pltpu.VMEM_SHARED((np_, d), jnp.bfloat16),
        pltpu.SemaphoreType.DMA,
        pltpu.SemaphoreType.DMA,
        pltpu.SemaphoreType.DMA,
        pltpu.SemaphoreType.DMA,
    ]
    if with_deg:
        out_type.append(jax.ShapeDtypeStruct((nacc, np_, 8), jnp.float32))
        scratch += [
            pltpu.VMEM((CHUNK, 8), jnp.float32),
            pltpu.VMEM_SHARED((np_, 8), jnp.float32),
            pltpu.VMEM((CHUNK, 8), jnp.float32),
        ]
    assert rows_per_tile % CHUNK == 0
    n_stage = rows_per_tile // CHUNK

    def body(z_hbm, srcs_hbm, dsts_hbm, zrow_hbm, zdeg_hbm, ones_hbm,
             acc_out, *rest):
        if with_deg:
            (deg_out, src_v, dst_v, rows0, rows1, acc_sh, sem0, sem1,
             ssem0, ssem1, ones_v, deg_sh, deg_st) = rest
        else:
            (src_v, dst_v, rows0, rows1, acc_sh, sem0, sem1,
             ssem0, ssem1) = rest
        bufs, sems, ssems = (rows0, rows1), (sem0, sem1), (ssem0, ssem1)
        c = lax.axis_index("c")
        s = lax.axis_index("s")
        my_nch = jnp.where(c == 0, nch0, nch1)
        chunk0 = jnp.where(c == 0, s * nch0, NS * nch0 + s * nch1)
        r0 = s * rows_per_tile
        out_c = jnp.minimum(c, nacc - 1)

        def work():
            pltpu.sync_copy(zrow_hbm.at[pl.ds(0, CHUNK)], rows0)
            if with_deg:
                pltpu.sync_copy(zdeg_hbm.at[pl.ds(0, CHUNK)], deg_st)

            zs = []
            for k in range(n_stage):
                blk_rows = pl.ds(r0 + k * CHUNK, CHUNK)
                zs.append(pltpu.async_copy(rows0, acc_sh.at[blk_rows], sem0))
                if with_deg:
                    zs.append(pltpu.async_copy(deg_st, deg_sh.at[blk_rows],
                                               sem1))
            if with_deg:
                pltpu.sync_copy(ones_hbm, ones_v)
            for h in zs:
                h.wait()
            plsc.subcore_barrier()

            def group_body(g, carry):
                base = chunk0 + g * grp
                pltpu.sync_copy(srcs_hbm.at[pl.ds(base, grp)], src_v)
                pltpu.sync_copy(dsts_hbm.at[pl.ds(base, grp)], dst_v)
                pend = [pltpu.async_copy(z_hbm.at[src_v.at[0]], rows0, sem0),
                        None]
                scat = [None, None]
                for j in range(grp):
                    b = j % 2
                    if j + 1 < grp:
                        nb = (j + 1) % 2
                        if scat[nb] is not None:
                            scat[nb].wait()
                        pend[nb] = pltpu.async_copy(
                            z_hbm.at[src_v.at[j + 1]], bufs[nb], sems[nb])
                    pend[b].wait()
                    scat[b] = pltpu.async_copy(
                        bufs[b], acc_sh.at[dst_v.at[j]], ssems[b], add=True)
                    if with_deg:
                        pltpu.sync_copy(ones_v, deg_sh.at[dst_v.at[j]],
                                        add=True)
                for h in scat:
                    if h is not None:
                        h.wait()
                return carry

            lax.fori_loop(0, my_nch // grp, group_body, 0)
            plsc.subcore_barrier()

            wr = [None, None]
            for k in range(n_stage):
                b = k % 2
                if wr[b] is not None:
                    wr[b].wait()
                blk_rows = pl.ds(r0 + k * CHUNK, CHUNK)
                pltpu.sync_copy(acc_sh.at[blk_rows], bufs[b])
                wr[b] = pltpu.async_copy(bufs[b], acc_out.at[out_c, blk_rows],
                                         sems[b])
                if with_deg:
                    pltpu.sync_copy(deg_sh.at[blk_rows], deg_st)
                    pltpu.sync_copy(deg_st, deg_out.at[out_c, blk_rows])
            for h in wr:
                if h is not None:
                    h.wait()

        if nacc == 1:
            pl.when(c == 0)(work)
        else:
            work()

    return pl.kernel(
        body, out_type=out_type, mesh=mesh, scratch_types=scratch,
        compiler_params=pltpu.CompilerParams(use_tc_tiling_on_sc=False),
    )


def kernel(x, edge_index, W_self0, W_neigh0, b0, W_self1, W_neigh1, b1):
    n, d = x.shape
    e = edge_index.shape[1]
    blk = 512

    np_ = ((n + 1 + NW * 8 - 1) // (NW * 8)) * (NW * 8)
    nch_pair = -(-e // (NS * CHUNK))
    nch_pair = -(-nch_pair // 16) * 16
    nch0 = nch1 = nch_pair // 2
    epad = NS * nch_pair * CHUNK

    pad_idx = n + (jnp.arange(epad - e, dtype=jnp.int32) % (np_ - n))
    src = edge_index[0].astype(jnp.int32)
    dst = edge_index[1].astype(jnp.int32)
    srcs = jnp.concatenate([src, pad_idx]).reshape(NS * nch_pair, CHUNK)
    dsts = jnp.concatenate([dst, pad_idx]).reshape(NS * nch_pair, CHUNK)
    zrow = jnp.zeros((np_, d), jnp.bfloat16)
    zdeg = jnp.zeros((np_, 8), jnp.float32)
    ones8 = jnp.ones((CHUNK, 8), jnp.float32)
    xp = jnp.pad(x, ((0, np_ - n), (0, 0)))

    edge_pass0 = _make_edge_pass(np_, d, nch0, nch1, with_deg=True)
    edge_pass1 = _make_edge_pass(np_, d, nch0, nch1, with_deg=False)

    zb = jnp.zeros((d,), jnp.float32)
    z0 = _mm(xp, W_neigh0, zb, jnp.bfloat16, blk)
    s0 = _mm(xp, W_self0, b0, jnp.float32, blk)
    acc0, deg = edge_pass0(z0, srcs, dsts, zrow, zdeg, ones8)
    z1 = _combine_mm(s0, acc0, deg, W_neigh1, zb, jnp.bfloat16, blk)
    s1 = _combine_mm(s0, acc0, deg, W_self1, b1, jnp.float32, blk)
    acc1 = edge_pass1(z1, srcs, dsts, zrow, zdeg, ones8)
    if isinstance(acc1, (list, tuple)):
        acc1 = acc1[0]
    return _combine(s1, acc1, deg, n, 400)


# speedup vs baseline: 1.2147x; 1.2147x over previous
import functools

import jax
import jax.numpy as jnp
from jax import lax
from jax.experimental import pallas as pl
from jax.experimental.pallas import tpu as pltpu
from jax.experimental.pallas import tpu_sc as plsc

NC = 2
NS = 16
NW = NC * NS
CHUNK = 128
NBUF = 4


def _sum_parts(ref):
    tot = ref[0]
    for k in range(1, ref.shape[0]):
        tot = tot + ref[k]
    return tot


def _mm2_body(x_ref, ws_ref, wn_ref, b_ref, s_ref, z_ref):
    x = x_ref[...]
    s_ref[...] = (
        jnp.dot(x, ws_ref[...], preferred_element_type=jnp.float32) + b_ref[...]
    )
    z_ref[...] = jnp.dot(
        x, wn_ref[...], preferred_element_type=jnp.float32
    ).astype(jnp.bfloat16)


def _combine_mm_body(s_ref, acc_ref, deg_ref, ws_ref, wn_ref, b_ref,
                     s_out_ref, z_out_ref):
    deg = _sum_parts(deg_ref)[:, 0:1]
    rdeg = 1.0 / jnp.maximum(deg, 1.0)
    acc = _sum_parts(acc_ref).astype(jnp.float32)
    h = jnp.maximum(s_ref[...] + acc * rdeg, 0.0)
    s_out_ref[...] = (
        jnp.dot(h, ws_ref[...], preferred_element_type=jnp.float32) + b_ref[...]
    )
    z_out_ref[...] = jnp.dot(
        h, wn_ref[...], preferred_element_type=jnp.float32
    ).astype(jnp.bfloat16)


def _combine_body(s_ref, acc_ref, deg_ref, o_ref):
    deg = _sum_parts(deg_ref)[:, 0:1]
    rdeg = 1.0 / jnp.maximum(deg, 1.0)
    acc = _sum_parts(acc_ref).astype(jnp.float32)
    o_ref[...] = jnp.maximum(s_ref[...] + acc * rdeg, 0.0)


def _mm2(x, np_, w_self, w_neigh, b, blk):
    d = x.shape[1]
    grid = np_ // blk
    return pl.pallas_call(
        _mm2_body,
        grid=(grid,),
        in_specs=[
            pl.BlockSpec((blk, d), lambda i: (i, 0)),
            pl.BlockSpec((d, d), lambda i: (0, 0)),
            pl.BlockSpec((d, d), lambda i: (0, 0)),
            pl.BlockSpec((1, d), lambda i: (0, 0)),
        ],
        out_specs=[
            pl.BlockSpec((blk, d), lambda i: (i, 0)),
            pl.BlockSpec((blk, d), lambda i: (i, 0)),
        ],
        out_shape=[
            jax.ShapeDtypeStruct((np_, d), jnp.float32),
            jax.ShapeDtypeStruct((np_, d), jnp.bfloat16),
        ],
    )(x, w_self, w_neigh, b.reshape(1, d))


def _combine_mm(s, acc, deg, w_self, w_neigh, b, blk):
    n, d = s.shape
    grid = n // blk
    na = acc.shape[0]
    return pl.pallas_call(
        _combine_mm_body,
        grid=(grid,),
        in_specs=[
            pl.BlockSpec((blk, d), lambda i: (i, 0)),
            pl.BlockSpec((na, blk, d), lambda i: (0, i, 0)),
            pl.BlockSpec((na, blk, 8), lambda i: (0, i, 0)),
            pl.BlockSpec((d, d), lambda i: (0, 0)),
            pl.BlockSpec((d, d), lambda i: (0, 0)),
            pl.BlockSpec((1, d), lambda i: (0, 0)),
        ],
        out_specs=[
            pl.BlockSpec((blk, d), lambda i: (i, 0)),
            pl.BlockSpec((blk, d), lambda i: (i, 0)),
        ],
        out_shape=[
            jax.ShapeDtypeStruct((n, d), jnp.float32),
            jax.ShapeDtypeStruct((n, d), jnp.bfloat16),
        ],
    )(s, acc, deg, w_self, w_neigh, b.reshape(1, d))


def _combine(s, acc, deg, n_out, blk):
    d = s.shape[1]
    grid = n_out // blk
    na = acc.shape[0]
    return pl.pallas_call(
        _combine_body,
        grid=(grid,),
        in_specs=[
            pl.BlockSpec((blk, d), lambda i: (i, 0)),
            pl.BlockSpec((na, blk, d), lambda i: (0, i, 0)),
            pl.BlockSpec((na, blk, 8), lambda i: (0, i, 0)),
        ],
        out_specs=pl.BlockSpec((blk, d), lambda i: (i, 0)),
        out_shape=jax.ShapeDtypeStruct((n_out, d), jnp.float32),
    )(s, acc, deg)


def _make_edge_pass(np_, d, nch, with_deg):
    rows_per_tile = np_ // NS
    assert nch % NBUF == 0 and nch // NBUF >= 2
    assert rows_per_tile % CHUNK == 0
    n_stage = rows_per_tile // CHUNK
    mesh = plsc.VectorSubcoreMesh(core_axis_name="c", subcore_axis_name="s")

    out_type = [jax.ShapeDtypeStruct((NC, np_, d), jnp.bfloat16)]
    scratch = [
        pltpu.VMEM((nch, CHUNK), jnp.int32),
        pltpu.VMEM((nch, CHUNK), jnp.int32),
    ]
    scratch += [pltpu.VMEM((CHUNK, d), jnp.bfloat16) for _ in range(NBUF)]
    scratch += [pltpu.VMEM_SHARED((np_, d), jnp.bfloat16)]
    scratch += [pltpu.SemaphoreType.DMA for _ in range(2 * NBUF)]
    if with_deg:
        out_type.append(jax.ShapeDtypeStruct((NC, np_, 8), jnp.float32))
        scratch += [
            pltpu.VMEM((CHUNK, 8), jnp.float32),
            pltpu.VMEM_SHARED((np_, 8), jnp.float32),
            pltpu.VMEM((CHUNK, 8), jnp.float32),
        ]

    def body(z_hbm, srcs_hbm, dsts_hbm, zrow_hbm, zdeg_hbm, ones_hbm,
             acc_out, *rest):
        if with_deg:
            deg_out = rest[0]
            rest = rest[1:]
            ones_v, deg_sh, deg_st = rest[-3:]
            rest = rest[:-3]
        src_v, dst_v = rest[0], rest[1]
        bufs = rest[2:2 + NBUF]
        acc_sh = rest[2 + NBUF]
        gs = rest[3 + NBUF:3 + 2 * NBUF]
        ss = rest[3 + 2 * NBUF:3 + 3 * NBUF]
        c = lax.axis_index("c")
        s = lax.axis_index("s")
        w = c * NS + s
        r0 = s * rows_per_tile

        pltpu.sync_copy(zrow_hbm.at[pl.ds(0, CHUNK)], bufs[0])
        if with_deg:
            pltpu.sync_copy(zdeg_hbm.at[pl.ds(0, CHUNK)], deg_st)
        zs = []
        for k in range(n_stage):
            blk_rows = pl.ds(r0 + k * CHUNK, CHUNK)
            zs.append(pltpu.async_copy(bufs[0], acc_sh.at[blk_rows], gs[0]))
            if with_deg:
                zs.append(pltpu.async_copy(deg_st, deg_sh.at[blk_rows], gs[1]))
        if with_deg:
            pltpu.sync_copy(ones_hbm, ones_v)
        pltpu.sync_copy(srcs_hbm.at[pl.ds(w * nch, nch)], src_v)
        pltpu.sync_copy(dsts_hbm.at[pl.ds(w * nch, nch)], dst_v)
        for h in zs:
            h.wait()
        plsc.subcore_barrier()

        def gather(k, j):
            return pltpu.async_copy(z_hbm.at[src_v.at[k]], bufs[j], gs[j])

        def wait_gather(j):
            pltpu.make_async_copy(z_hbm.at[src_v.at[0]], bufs[j],
                                  gs[j]).wait()

        def process(k, j):
            wait_gather(j)
            h = pltpu.async_copy(bufs[j], acc_sh.at[dst_v.at[k]], ss[j],
                                 add=True)
            if with_deg:
                pltpu.sync_copy(ones_v, deg_sh.at[dst_v.at[k]], add=True)
            return h

        for j in range(NBUF):
            gather(j, j)

        def quad_body(q, carry):
            a = q * NBUF
            h0 = process(a + 0, 0)
            h1 = process(a + 1, 1)
            h0.wait()
            gather(a + NBUF + 0, 0)
            h2 = process(a + 2, 2)
            h1.wait()
            gather(a + NBUF + 1, 1)
            h3 = process(a + 3, 3)
            h2.wait()
            gather(a + NBUF + 2, 2)
            h3.wait()
            gather(a + NBUF + 3, 3)
            return carry

        lax.fori_loop(0, nch // NBUF - 1, quad_body, 0)
        a0 = nch - NBUF
        tails = [process(a0 + j, j) for j in range(NBUF)]
        for h in tails:
            h.wait()
        plsc.subcore_barrier()

        wr = [None, None]
        for k in range(n_stage):
            b = k % 2
            if wr[b] is not None:
                wr[b].wait()
            blk_rows = pl.ds(r0 + k * CHUNK, CHUNK)
            pltpu.sync_copy(acc_sh.at[blk_rows], bufs[b])
            wr[b] = pltpu.async_copy(bufs[b], acc_out.at[c, blk_rows], gs[b])
            if with_deg:
                pltpu.sync_copy(deg_sh.at[blk_rows], deg_st)
                pltpu.sync_copy(deg_st, deg_out.at[c, blk_rows])
        for h in wr:
            if h is not None:
                h.wait()

    return pl.kernel(
        body, out_type=out_type, mesh=mesh, scratch_types=scratch,
        compiler_params=pltpu.CompilerParams(use_tc_tiling_on_sc=False),
    )


def kernel(x, edge_index, W_self0, W_neigh0, b0, W_self1, W_neigh1, b1):
    n, d = x.shape
    e = edge_index.shape[1]
    blk = 512

    np_ = ((n + 1 + NW * 8 - 1) // (NW * 8)) * (NW * 8)
    nch = -(-e // (NW * CHUNK))
    nch = -(-nch // (2 * NBUF)) * (2 * NBUF)
    epad = NW * nch * CHUNK

    pad_idx = n + (jnp.arange(epad - e, dtype=jnp.int32) % (np_ - n))
    src = edge_index[0].astype(jnp.int32)
    dst = edge_index[1].astype(jnp.int32)
    srcs = jnp.concatenate([src, pad_idx]).reshape(NW * nch, CHUNK)
    dsts = jnp.concatenate([dst, pad_idx]).reshape(NW * nch, CHUNK)
    zrow = jnp.zeros((np_, d), jnp.bfloat16)
    zdeg = jnp.zeros((np_, 8), jnp.float32)
    ones8 = jnp.ones((CHUNK, 8), jnp.float32)

    edge_pass0 = _make_edge_pass(np_, d, nch, with_deg=True)
    edge_pass1 = _make_edge_pass(np_, d, nch, with_deg=False)

    s0, z0 = _mm2(x, np_, W_self0, W_neigh0, b0, blk)
    acc0, deg = edge_pass0(z0, srcs, dsts, zrow, zdeg, ones8)
    s1, z1 = _combine_mm(s0, acc0, deg, W_self1, W_neigh1, b1, blk)
    acc1 = edge_pass1(z1, srcs, dsts, zrow, zdeg, ones8)
    if isinstance(acc1, (list, tuple)):
        acc1 = acc1[0]
    return _combine(s1, acc1, deg, n, 400)


# speedup vs baseline: 1.3021x; 1.0720x over previous
import functools

import jax
import jax.numpy as jnp
from jax import lax
from jax.experimental import pallas as pl
from jax.experimental.pallas import tpu as pltpu
from jax.experimental.pallas import tpu_sc as plsc

NC = 2
NS = 16
NW = NC * NS
CHUNK = 128
NBUF = 4


def _sum_parts(ref):
    tot = ref[0]
    for k in range(1, ref.shape[0]):
        tot = tot + ref[k]
    return tot


def _mm2_body(x_ref, ws_ref, wn_ref, b_ref, s_ref, z_ref):
    x = x_ref[...]
    s_ref[...] = (
        jnp.dot(x, ws_ref[...], preferred_element_type=jnp.float32) + b_ref[...]
    )
    z_ref[...] = jnp.dot(
        x, wn_ref[...], preferred_element_type=jnp.float32
    ).astype(jnp.bfloat16)


def _combine_mm_body(s_ref, acc_ref, deg_ref, ws_ref, wn_ref, b_ref,
                     s_out_ref, z_out_ref):
    deg = _sum_parts(deg_ref)[:, 0:1]
    rdeg = 1.0 / jnp.maximum(deg, 1.0)
    acc = _sum_parts(acc_ref).astype(jnp.float32)
    h = jnp.maximum(s_ref[...] + acc * rdeg, 0.0)
    s_out_ref[...] = (
        jnp.dot(h, ws_ref[...], preferred_element_type=jnp.float32) + b_ref[...]
    )
    z_out_ref[...] = jnp.dot(
        h, wn_ref[...], preferred_element_type=jnp.float32
    ).astype(jnp.bfloat16)


def _combine_body(s_ref, acc_ref, deg_ref, o_ref):
    deg = _sum_parts(deg_ref)[:, 0:1]
    rdeg = 1.0 / jnp.maximum(deg, 1.0)
    acc = _sum_parts(acc_ref).astype(jnp.float32)
    o_ref[...] = jnp.maximum(s_ref[...] + acc * rdeg, 0.0)


def _mm2(x, np_, w_self, w_neigh, b, blk):
    d = x.shape[1]
    grid = np_ // blk
    return pl.pallas_call(
        _mm2_body,
        grid=(grid,),
        in_specs=[
            pl.BlockSpec((blk, d), lambda i: (i, 0)),
            pl.BlockSpec((d, d), lambda i: (0, 0)),
            pl.BlockSpec((d, d), lambda i: (0, 0)),
            pl.BlockSpec((1, d), lambda i: (0, 0)),
        ],
        out_specs=[
            pl.BlockSpec((blk, d), lambda i: (i, 0)),
            pl.BlockSpec((blk, d), lambda i: (i, 0)),
        ],
        out_shape=[
            jax.ShapeDtypeStruct((np_, d), jnp.float32),
            jax.ShapeDtypeStruct((np_, d), jnp.bfloat16),
        ],
    )(x, w_self, w_neigh, b.reshape(1, d))


def _combine_mm(s, acc, deg, w_self, w_neigh, b, blk):
    n, d = s.shape
    grid = n // blk
    na = acc.shape[0]
    return pl.pallas_call(
        _combine_mm_body,
        grid=(grid,),
        in_specs=[
            pl.BlockSpec((blk, d), lambda i: (i, 0)),
            pl.BlockSpec((na, blk, d), lambda i: (0, i, 0)),
            pl.BlockSpec((na, blk, 8), lambda i: (0, i, 0)),
            pl.BlockSpec((d, d), lambda i: (0, 0)),
            pl.BlockSpec((d, d), lambda i: (0, 0)),
            pl.BlockSpec((1, d), lambda i: (0, 0)),
        ],
        out_specs=[
            pl.BlockSpec((blk, d), lambda i: (i, 0)),
            pl.BlockSpec((blk, d), lambda i: (i, 0)),
        ],
        out_shape=[
            jax.ShapeDtypeStruct((n, d), jnp.float32),
            jax.ShapeDtypeStruct((n, d), jnp.bfloat16),
        ],
    )(s, acc, deg, w_self, w_neigh, b.reshape(1, d))


def _combine(s, acc, deg, n_out, blk):
    d = s.shape[1]
    grid = n_out // blk
    na = acc.shape[0]
    return pl.pallas_call(
        _combine_body,
        grid=(grid,),
        in_specs=[
            pl.BlockSpec((blk, d), lambda i: (i, 0)),
            pl.BlockSpec((na, blk, d), lambda i: (0, i, 0)),
            pl.BlockSpec((na, blk, 8), lambda i: (0, i, 0)),
        ],
        out_specs=pl.BlockSpec((blk, d), lambda i: (i, 0)),
        out_shape=jax.ShapeDtypeStruct((n_out, d), jnp.float32),
    )(s, acc, deg)


def _make_edge_pass(np_, d, nch, with_deg):
    rows_per_tile = np_ // NS
    assert nch % NBUF == 0 and nch // NBUF >= 2
    assert rows_per_tile % CHUNK == 0
    n_stage = rows_per_tile // CHUNK
    mesh = plsc.VectorSubcoreMesh(core_axis_name="c", subcore_axis_name="s")

    out_type = [jax.ShapeDtypeStruct((NC, np_, d), jnp.bfloat16)]
    scratch = [
        pltpu.VMEM((nch, CHUNK), jnp.int32),
        pltpu.VMEM((nch, CHUNK), jnp.int32),
    ]
    scratch += [pltpu.VMEM((CHUNK, d), jnp.bfloat16) for _ in range(NBUF)]
    scratch += [pltpu.VMEM_SHARED((np_, d), jnp.bfloat16)]
    scratch += [pltpu.SemaphoreType.DMA for _ in range(2 * NBUF)]
    if with_deg:
        out_type.append(jax.ShapeDtypeStruct((NC, np_, 8), jnp.float32))
        scratch += [
            pltpu.VMEM((CHUNK, 8), jnp.float32),
            pltpu.VMEM_SHARED((np_, 8), jnp.float32),
            pltpu.VMEM((CHUNK, 8), jnp.float32),
        ]

    def body(z_hbm, srcs_hbm, dsts_hbm, zrow_hbm, zdeg_hbm, ones_hbm,
             acc_out, *rest):
        if with_deg:
            deg_out = rest[0]
            rest = rest[1:]
            ones_v, deg_sh, deg_st = rest[-3:]
            rest = rest[:-3]
        src_v, dst_v = rest[0], rest[1]
        bufs = rest[2:2 + NBUF]
        acc_sh = rest[2 + NBUF]
        gs = rest[3 + NBUF:3 + 2 * NBUF]
        ss = rest[3 + 2 * NBUF:3 + 3 * NBUF]
        c = lax.axis_index("c")
        s = lax.axis_index("s")
        w = c * NS + s
        r0 = s * rows_per_tile

        pltpu.sync_copy(zrow_hbm.at[pl.ds(0, CHUNK)], bufs[0])
        if with_deg:
            pltpu.sync_copy(zdeg_hbm.at[pl.ds(0, CHUNK)], deg_st)
        zs = []
        for k in range(n_stage):
            blk_rows = pl.ds(r0 + k * CHUNK, CHUNK)
            zs.append(pltpu.async_copy(bufs[0], acc_sh.at[blk_rows], gs[0]))
            if with_deg:
                zs.append(pltpu.async_copy(deg_st, deg_sh.at[blk_rows], gs[1]))
        if with_deg:
            pltpu.sync_copy(ones_hbm, ones_v)
        pltpu.sync_copy(srcs_hbm.at[pl.ds(w * nch, nch)], src_v)
        pltpu.sync_copy(dsts_hbm.at[pl.ds(w * nch, nch)], dst_v)
        for h in zs:
            h.wait()
        plsc.subcore_barrier()

        def gather(k, j):
            return pltpu.async_copy(z_hbm.at[src_v.at[k]], bufs[j], gs[j])

        def wait_gather(j):
            pltpu.make_async_copy(z_hbm.at[src_v.at[0]], bufs[j],
                                  gs[j]).wait()

        def process(k, j):
            wait_gather(j)
            h = pltpu.async_copy(bufs[j], acc_sh.at[dst_v.at[k]], ss[j],
                                 add=True)
            if with_deg:
                pltpu.sync_copy(ones_v, deg_sh.at[dst_v.at[k]], add=True)
            return h

        for j in range(NBUF):
            gather(j, j)

        def quad_body(q, carry):
            a = q * NBUF
            h0 = process(a + 0, 0)
            h1 = process(a + 1, 1)
            h0.wait()
            gather(a + NBUF + 0, 0)
            h2 = process(a + 2, 2)
            h1.wait()
            gather(a + NBUF + 1, 1)
            h3 = process(a + 3, 3)
            h2.wait()
            gather(a + NBUF + 2, 2)
            h3.wait()
            gather(a + NBUF + 3, 3)
            return carry

        lax.fori_loop(0, nch // NBUF - 1, quad_body, 0)
        a0 = nch - NBUF
        tails = [process(a0 + j, j) for j in range(NBUF)]
        for h in tails:
            h.wait()
        plsc.subcore_barrier()

        wr = [None, None]
        for k in range(n_stage):
            b = k % 2
            if wr[b] is not None:
                wr[b].wait()
            blk_rows = pl.ds(r0 + k * CHUNK, CHUNK)
            pltpu.sync_copy(acc_sh.at[blk_rows], bufs[b])
            wr[b] = pltpu.async_copy(bufs[b], acc_out.at[c, blk_rows], gs[b])
            if with_deg:
                pltpu.sync_copy(deg_sh.at[blk_rows], deg_st)
                pltpu.sync_copy(deg_st, deg_out.at[c, blk_rows])
        for h in wr:
            if h is not None:
                h.wait()

    return pl.kernel(
        body, out_type=out_type, mesh=mesh, scratch_types=scratch,
        compiler_params=pltpu.CompilerParams(use_tc_tiling_on_sc=False),
    )


def kernel(x, edge_index, W_self0, W_neigh0, b0, W_self1, W_neigh1, b1):
    n, d = x.shape
    e = edge_index.shape[1]
    blk = 1024

    np_ = ((n + 1 + NW * 8 - 1) // (NW * 8)) * (NW * 8)
    nch = -(-e // (NW * CHUNK))
    nch = -(-nch // (2 * NBUF)) * (2 * NBUF)
    epad = NW * nch * CHUNK

    pad_idx = n + (jnp.arange(epad - e, dtype=jnp.int32) % (np_ - n))
    src = edge_index[0].astype(jnp.int32)
    dst = edge_index[1].astype(jnp.int32)
    srcs = jnp.concatenate([src, pad_idx]).reshape(NW * nch, CHUNK)
    dsts = jnp.concatenate([dst, pad_idx]).reshape(NW * nch, CHUNK)
    zrow = jnp.zeros((np_, d), jnp.bfloat16)
    zdeg = jnp.zeros((np_, 8), jnp.float32)
    ones8 = jnp.ones((CHUNK, 8), jnp.float32)

    edge_pass0 = _make_edge_pass(np_, d, nch, with_deg=True)
    edge_pass1 = _make_edge_pass(np_, d, nch, with_deg=False)

    s0, z0 = _mm2(x, np_, W_self0, W_neigh0, b0, blk)
    acc0, deg = edge_pass0(z0, srcs, dsts, zrow, zdeg, ones8)
    s1, z1 = _combine_mm(s0, acc0, deg, W_self1, W_neigh1, b1, blk)
    acc1 = edge_pass1(z1, srcs, dsts, zrow, zdeg, ones8)
    if isinstance(acc1, (list, tuple)):
        acc1 = acc1[0]
    return _combine(s1, acc1, deg, n, 1000)


# speedup vs baseline: 1.3323x; 1.0232x over previous
import functools

import jax
import jax.numpy as jnp
from jax import lax
from jax.experimental import pallas as pl
from jax.experimental.pallas import tpu as pltpu
from jax.experimental.pallas import tpu_sc as plsc

NC = 2
NS = 16
NW = NC * NS
CHUNK = 128
NBUF = 4


def _sum_parts(ref):
    tot = ref[0]
    for k in range(1, ref.shape[0]):
        tot = tot + ref[k]
    return tot


def _mm2_body(x_ref, ws_ref, wn_ref, b_ref, s_ref, z_ref):
    x = x_ref[...]
    s_ref[...] = (
        jnp.dot(x, ws_ref[...], preferred_element_type=jnp.float32) + b_ref[...]
    )
    z_ref[...] = jnp.dot(
        x, wn_ref[...], preferred_element_type=jnp.float32
    ).astype(jnp.bfloat16)


def _combine_mm_body(s_ref, acc_ref, deg_ref, ws_ref, wn_ref, b_ref,
                     s_out_ref, z_out_ref):
    deg = _sum_parts(deg_ref)[:, 0:1]
    rdeg = 1.0 / jnp.maximum(deg, 1.0)
    acc = _sum_parts(acc_ref).astype(jnp.float32)
    h = jnp.maximum(s_ref[...] + acc * rdeg, 0.0)
    s_out_ref[...] = (
        jnp.dot(h, ws_ref[...], preferred_element_type=jnp.float32) + b_ref[...]
    )
    z_out_ref[...] = jnp.dot(
        h, wn_ref[...], preferred_element_type=jnp.float32
    ).astype(jnp.bfloat16)


def _combine_body(s_ref, acc_ref, deg_ref, o_ref):
    deg = _sum_parts(deg_ref)[:, 0:1]
    rdeg = 1.0 / jnp.maximum(deg, 1.0)
    acc = _sum_parts(acc_ref).astype(jnp.float32)
    o_ref[...] = jnp.maximum(s_ref[...] + acc * rdeg, 0.0)


def _mm2(x, np_, w_self, w_neigh, b, blk):
    d = x.shape[1]
    grid = np_ // blk
    return pl.pallas_call(
        _mm2_body,
        grid=(grid,),
        in_specs=[
            pl.BlockSpec((blk, d), lambda i: (i, 0)),
            pl.BlockSpec((d, d), lambda i: (0, 0)),
            pl.BlockSpec((d, d), lambda i: (0, 0)),
            pl.BlockSpec((1, d), lambda i: (0, 0)),
        ],
        out_specs=[
            pl.BlockSpec((blk, d), lambda i: (i, 0)),
            pl.BlockSpec((blk, d), lambda i: (i, 0)),
        ],
        out_shape=[
            jax.ShapeDtypeStruct((np_, d), jnp.float32),
            jax.ShapeDtypeStruct((np_, d), jnp.bfloat16),
        ],
    )(x, w_self, w_neigh, b.reshape(1, d))


def _combine_mm(s, acc, deg, w_self, w_neigh, b, blk):
    n, d = s.shape
    grid = n // blk
    na = acc.shape[0]
    return pl.pallas_call(
        _combine_mm_body,
        grid=(grid,),
        in_specs=[
            pl.BlockSpec((blk, d), lambda i: (i, 0)),
            pl.BlockSpec((na, blk, d), lambda i: (0, i, 0)),
            pl.BlockSpec((na, blk, 8), lambda i: (0, i, 0)),
            pl.BlockSpec((d, d), lambda i: (0, 0)),
            pl.BlockSpec((d, d), lambda i: (0, 0)),
            pl.BlockSpec((1, d), lambda i: (0, 0)),
        ],
        out_specs=[
            pl.BlockSpec((blk, d), lambda i: (i, 0)),
            pl.BlockSpec((blk, d), lambda i: (i, 0)),
        ],
        out_shape=[
            jax.ShapeDtypeStruct((n, d), jnp.float32),
            jax.ShapeDtypeStruct((n, d), jnp.bfloat16),
        ],
    )(s, acc, deg, w_self, w_neigh, b.reshape(1, d))


def _combine(s, acc, deg, n_out, blk):
    d = s.shape[1]
    grid = n_out // blk
    na = acc.shape[0]
    return pl.pallas_call(
        _combine_body,
        grid=(grid,),
        in_specs=[
            pl.BlockSpec((blk, d), lambda i: (i, 0)),
            pl.BlockSpec((na, blk, d), lambda i: (0, i, 0)),
            pl.BlockSpec((na, blk, 8), lambda i: (0, i, 0)),
        ],
        out_specs=pl.BlockSpec((blk, d), lambda i: (i, 0)),
        out_shape=jax.ShapeDtypeStruct((n_out, d), jnp.float32),
    )(s, acc, deg)


def _make_edge_pass(np_, d, nch, with_deg):
    rows_per_tile = np_ // NS
    assert nch % NBUF == 0 and nch // NBUF >= 2
    assert rows_per_tile % CHUNK == 0
    n_stage = rows_per_tile // CHUNK
    mesh = plsc.VectorSubcoreMesh(core_axis_name="c", subcore_axis_name="s")

    out_type = [jax.ShapeDtypeStruct((NC, np_, d), jnp.bfloat16)]
    scratch = [
        pltpu.VMEM((nch, CHUNK), jnp.int32),
        pltpu.VMEM((nch, CHUNK), jnp.int32),
    ]
    scratch += [pltpu.VMEM((CHUNK, d), jnp.bfloat16) for _ in range(NBUF)]
    scratch += [pltpu.VMEM_SHARED((np_, d), jnp.bfloat16)]
    scratch += [pltpu.SemaphoreType.DMA for _ in range(2 * NBUF)]
    if with_deg:
        out_type.append(jax.ShapeDtypeStruct((NC, np_, 8), jnp.float32))
        scratch += [
            pltpu.VMEM((CHUNK, 8), jnp.float32),
            pltpu.VMEM_SHARED((np_, 8), jnp.float32),
            pltpu.VMEM((CHUNK, 8), jnp.float32),
        ]

    def body(z_hbm, srcs_hbm, dsts_hbm, zrow_hbm, zdeg_hbm, ones_hbm,
             acc_out, *rest):
        if with_deg:
            deg_out = rest[0]
            rest = rest[1:]
            ones_v, deg_sh, deg_st = rest[-3:]
            rest = rest[:-3]
        src_v, dst_v = rest[0], rest[1]
        bufs = rest[2:2 + NBUF]
        acc_sh = rest[2 + NBUF]
        gs = rest[3 + NBUF:3 + 2 * NBUF]
        ss = rest[3 + 2 * NBUF:3 + 3 * NBUF]
        c = lax.axis_index("c")
        s = lax.axis_index("s")
        w = c * NS + s
        r0 = s * rows_per_tile

        pltpu.sync_copy(zrow_hbm.at[pl.ds(0, CHUNK)], bufs[0])
        if with_deg:
            pltpu.sync_copy(zdeg_hbm.at[pl.ds(0, CHUNK)], deg_st)
        zs = []
        for k in range(n_stage):
            blk_rows = pl.ds(r0 + k * CHUNK, CHUNK)
            zs.append(pltpu.async_copy(bufs[0], acc_sh.at[blk_rows], gs[0]))
            if with_deg:
                zs.append(pltpu.async_copy(deg_st, deg_sh.at[blk_rows], gs[1]))
        if with_deg:
            pltpu.sync_copy(ones_hbm, ones_v)
        pltpu.sync_copy(srcs_hbm.at[pl.ds(w * nch, nch)], src_v)
        pltpu.sync_copy(dsts_hbm.at[pl.ds(w * nch, nch)], dst_v)
        for h in zs:
            h.wait()
        plsc.subcore_barrier()

        def gather(k, j):
            return pltpu.async_copy(z_hbm.at[src_v.at[k]], bufs[j], gs[j])

        def wait_gather(j):
            pltpu.make_async_copy(z_hbm.at[src_v.at[0]], bufs[j],
                                  gs[j]).wait()

        def process(k, j):
            wait_gather(j)
            h = pltpu.async_copy(bufs[j], acc_sh.at[dst_v.at[k]], ss[j],
                                 add=True)
            if with_deg:
                pltpu.sync_copy(ones_v, deg_sh.at[dst_v.at[k]], add=True)
            return h

        for j in range(NBUF):
            gather(j, j)

        def quad_body(q, carry):
            a = q * NBUF
            h0 = process(a + 0, 0)
            h1 = process(a + 1, 1)
            h0.wait()
            gather(a + NBUF + 0, 0)
            h2 = process(a + 2, 2)
            h1.wait()
            gather(a + NBUF + 1, 1)
            h3 = process(a + 3, 3)
            h2.wait()
            gather(a + NBUF + 2, 2)
            h3.wait()
            gather(a + NBUF + 3, 3)
            return carry

        lax.fori_loop(0, nch // NBUF - 1, quad_body, 0)
        a0 = nch - NBUF
        tails = [process(a0 + j, j) for j in range(NBUF)]
        for h in tails:
            h.wait()
        plsc.subcore_barrier()

        wr = [None, None]
        for k in range(n_stage):
            b = k % 2
            if wr[b] is not None:
                wr[b].wait()
            blk_rows = pl.ds(r0 + k * CHUNK, CHUNK)
            pltpu.sync_copy(acc_sh.at[blk_rows], bufs[b])
            wr[b] = pltpu.async_copy(bufs[b], acc_out.at[c, blk_rows], gs[b])
            if with_deg:
                pltpu.sync_copy(deg_sh.at[blk_rows], deg_st)
                pltpu.sync_copy(deg_st, deg_out.at[c, blk_rows])
        for h in wr:
            if h is not None:
                h.wait()

    return pl.kernel(
        body, out_type=out_type, mesh=mesh, scratch_types=scratch,
        compiler_params=pltpu.CompilerParams(use_tc_tiling_on_sc=False),
    )


def kernel(x, edge_index, W_self0, W_neigh0, b0, W_self1, W_neigh1, b1):
    n, d = x.shape
    e = edge_index.shape[1]
    blk = 2048

    np_ = ((n + 1 + NW * 8 - 1) // (NW * 8)) * (NW * 8)
    nch = -(-e // (NW * CHUNK))
    nch = -(-nch // (2 * NBUF)) * (2 * NBUF)
    epad = NW * nch * CHUNK

    pad_idx = n + (jnp.arange(epad - e, dtype=jnp.int32) % (np_ - n))
    src = edge_index[0].astype(jnp.int32)
    dst = edge_index[1].astype(jnp.int32)
    srcs = jnp.concatenate([src, pad_idx]).reshape(NW * nch, CHUNK)
    dsts = jnp.concatenate([dst, pad_idx]).reshape(NW * nch, CHUNK)
    zrow = jnp.zeros((np_, d), jnp.bfloat16)
    zdeg = jnp.zeros((np_, 8), jnp.float32)
    ones8 = jnp.ones((CHUNK, 8), jnp.float32)

    edge_pass0 = _make_edge_pass(np_, d, nch, with_deg=True)
    edge_pass1 = _make_edge_pass(np_, d, nch, with_deg=False)

    s0, z0 = _mm2(x, np_, W_self0, W_neigh0, b0, blk)
    acc0, deg = edge_pass0(z0, srcs, dsts, zrow, zdeg, ones8)
    s1, z1 = _combine_mm(s0, acc0, deg, W_self1, W_neigh1, b1, blk)
    acc1 = edge_pass1(z1, srcs, dsts, zrow, zdeg, ones8)
    if isinstance(acc1, (list, tuple)):
        acc1 = acc1[0]
    return _combine(s1, acc1, deg, n, 2000)


# speedup vs baseline: 1.3401x; 1.0058x over previous
import functools

import jax
import jax.numpy as jnp
from jax import lax
from jax.experimental import pallas as pl
from jax.experimental.pallas import tpu as pltpu
from jax.experimental.pallas import tpu_sc as plsc

NC = 2
NS = 16
NW = NC * NS
CHUNK = 128
NBUF = 4


def _sum_parts(ref):
    tot = ref[0]
    for k in range(1, ref.shape[0]):
        tot = tot + ref[k]
    return tot


def _mm2_body(x_ref, ws_ref, wn_ref, b_ref, s_ref, z_ref):
    x = x_ref[...]
    s_ref[...] = (
        jnp.dot(x, ws_ref[...], preferred_element_type=jnp.float32) + b_ref[...]
    )
    z_ref[...] = jnp.dot(
        x, wn_ref[...], preferred_element_type=jnp.float32
    ).astype(jnp.bfloat16)


def _combine_mm_body(s_ref, acc_ref, deg_ref, ws_ref, wn_ref, b_ref,
                     s_out_ref, z_out_ref):
    deg = _sum_parts(deg_ref)[:, 0:1]
    rdeg = 1.0 / jnp.maximum(deg, 1.0)
    acc = _sum_parts(acc_ref).astype(jnp.float32)
    h = jnp.maximum(s_ref[...] + acc * rdeg, 0.0)
    s_out_ref[...] = (
        jnp.dot(h, ws_ref[...], preferred_element_type=jnp.float32) + b_ref[...]
    )
    z_out_ref[...] = jnp.dot(
        h, wn_ref[...], preferred_element_type=jnp.float32
    ).astype(jnp.bfloat16)


def _combine_body(s_ref, acc_ref, deg_ref, o_ref):
    deg = _sum_parts(deg_ref)[:, 0:1]
    rdeg = 1.0 / jnp.maximum(deg, 1.0)
    acc = _sum_parts(acc_ref).astype(jnp.float32)
    o_ref[...] = jnp.maximum(s_ref[...] + acc * rdeg, 0.0)


def _mm2(x, np_, w_self, w_neigh, b, blk):
    d = x.shape[1]
    grid = np_ // blk
    return pl.pallas_call(
        _mm2_body,
        grid=(grid,),
        in_specs=[
            pl.BlockSpec((blk, d), lambda i: (i, 0)),
            pl.BlockSpec((d, d), lambda i: (0, 0)),
            pl.BlockSpec((d, d), lambda i: (0, 0)),
            pl.BlockSpec((1, d), lambda i: (0, 0)),
        ],
        out_specs=[
            pl.BlockSpec((blk, d), lambda i: (i, 0)),
            pl.BlockSpec((blk, d), lambda i: (i, 0)),
        ],
        out_shape=[
            jax.ShapeDtypeStruct((np_, d), jnp.float32),
            jax.ShapeDtypeStruct((np_, d), jnp.bfloat16),
        ],
    )(x, w_self, w_neigh, b.reshape(1, d))


def _combine_mm(s, acc, deg, w_self, w_neigh, b, blk):
    n, d = s.shape
    grid = n // blk
    na = acc.shape[0]
    return pl.pallas_call(
        _combine_mm_body,
        grid=(grid,),
        in_specs=[
            pl.BlockSpec((blk, d), lambda i: (i, 0)),
            pl.BlockSpec((na, blk, d), lambda i: (0, i, 0)),
            pl.BlockSpec((na, blk, 8), lambda i: (0, i, 0)),
            pl.BlockSpec((d, d), lambda i: (0, 0)),
            pl.BlockSpec((d, d), lambda i: (0, 0)),
            pl.BlockSpec((1, d), lambda i: (0, 0)),
        ],
        out_specs=[
            pl.BlockSpec((blk, d), lambda i: (i, 0)),
            pl.BlockSpec((blk, d), lambda i: (i, 0)),
        ],
        out_shape=[
            jax.ShapeDtypeStruct((n, d), jnp.float32),
            jax.ShapeDtypeStruct((n, d), jnp.bfloat16),
        ],
    )(s, acc, deg, w_self, w_neigh, b.reshape(1, d))


def _combine(s, acc, deg, n_out, blk):
    d = s.shape[1]
    grid = n_out // blk
    na = acc.shape[0]
    return pl.pallas_call(
        _combine_body,
        grid=(grid,),
        in_specs=[
            pl.BlockSpec((blk, d), lambda i: (i, 0)),
            pl.BlockSpec((na, blk, d), lambda i: (0, i, 0)),
            pl.BlockSpec((na, blk, 8), lambda i: (0, i, 0)),
        ],
        out_specs=pl.BlockSpec((blk, d), lambda i: (i, 0)),
        out_shape=jax.ShapeDtypeStruct((n_out, d), jnp.float32),
    )(s, acc, deg)


def _make_edge_pass(np_, d, nch, with_deg):
    rows_per_tile = np_ // NS
    assert nch % NBUF == 0 and nch // NBUF >= 2
    assert rows_per_tile % CHUNK == 0
    n_stage = rows_per_tile // CHUNK
    mesh = plsc.VectorSubcoreMesh(core_axis_name="c", subcore_axis_name="s")

    out_type = [jax.ShapeDtypeStruct((NC, np_, d), jnp.bfloat16)]
    scratch = [
        pltpu.VMEM((nch, CHUNK), jnp.int32),
        pltpu.VMEM((nch, CHUNK), jnp.int32),
    ]
    scratch += [pltpu.VMEM((CHUNK, d), jnp.bfloat16) for _ in range(NBUF)]
    scratch += [pltpu.VMEM_SHARED((np_, d), jnp.bfloat16)]
    scratch += [pltpu.SemaphoreType.DMA for _ in range(2 * NBUF)]
    if with_deg:
        out_type.append(jax.ShapeDtypeStruct((NC, np_, 8), jnp.float32))
        scratch += [
            pltpu.VMEM((CHUNK, 8), jnp.float32),
            pltpu.VMEM_SHARED((np_, 8), jnp.float32),
            pltpu.VMEM((CHUNK, 8), jnp.float32),
        ]

    def body(z_hbm, srcs_hbm, dsts_hbm, zrow_hbm, zdeg_hbm, ones_hbm,
             acc_out, *rest):
        if with_deg:
            deg_out = rest[0]
            rest = rest[1:]
            ones_v, deg_sh, deg_st = rest[-3:]
            rest = rest[:-3]
        src_v, dst_v = rest[0], rest[1]
        bufs = rest[2:2 + NBUF]
        acc_sh = rest[2 + NBUF]
        gs = rest[3 + NBUF:3 + 2 * NBUF]
        ss = rest[3 + 2 * NBUF:3 + 3 * NBUF]
        c = lax.axis_index("c")
        s = lax.axis_index("s")
        w = c * NS + s
        r0 = s * rows_per_tile

        pltpu.sync_copy(zrow_hbm.at[pl.ds(0, CHUNK)], bufs[0])
        if with_deg:
            pltpu.sync_copy(zdeg_hbm.at[pl.ds(0, CHUNK)], deg_st)
        zs = []
        for k in range(n_stage):
            blk_rows = pl.ds(r0 + k * CHUNK, CHUNK)
            zs.append(pltpu.async_copy(bufs[0], acc_sh.at[blk_rows], gs[0]))
            if with_deg:
                zs.append(pltpu.async_copy(deg_st, deg_sh.at[blk_rows], gs[1]))
        if with_deg:
            pltpu.sync_copy(ones_hbm, ones_v)
        pltpu.sync_copy(srcs_hbm.at[pl.ds(w * nch, nch)], src_v)
        pltpu.sync_copy(dsts_hbm.at[pl.ds(w * nch, nch)], dst_v)
        for h in zs:
            h.wait()
        plsc.subcore_barrier()

        def gather(k, j):
            return pltpu.async_copy(z_hbm.at[src_v.at[k]], bufs[j], gs[j])

        def wait_gather(j):
            pltpu.make_async_copy(z_hbm.at[src_v.at[0]], bufs[j],
                                  gs[j]).wait()

        def process(k, j):
            wait_gather(j)
            h = pltpu.async_copy(bufs[j], acc_sh.at[dst_v.at[k]], ss[j],
                                 add=True)
            if with_deg:
                pltpu.sync_copy(ones_v, deg_sh.at[dst_v.at[k]], add=True)
            return h

        for j in range(NBUF):
            gather(j, j)

        def quad_body(q, carry):
            a = q * NBUF
            h0 = process(a + 0, 0)
            h1 = process(a + 1, 1)
            h0.wait()
            gather(a + NBUF + 0, 0)
            h2 = process(a + 2, 2)
            h1.wait()
            gather(a + NBUF + 1, 1)
            h3 = process(a + 3, 3)
            h2.wait()
            gather(a + NBUF + 2, 2)
            h3.wait()
            gather(a + NBUF + 3, 3)
            return carry

        lax.fori_loop(0, nch // NBUF - 1, quad_body, 0)
        a0 = nch - NBUF
        tails = [process(a0 + j, j) for j in range(NBUF)]
        for h in tails:
            h.wait()
        plsc.subcore_barrier()

        wr = [None, None]
        for k in range(n_stage):
            b = k % 2
            if wr[b] is not None:
                wr[b].wait()
            blk_rows = pl.ds(r0 + k * CHUNK, CHUNK)
            pltpu.sync_copy(acc_sh.at[blk_rows], bufs[b])
            wr[b] = pltpu.async_copy(bufs[b], acc_out.at[c, blk_rows], gs[b])
            if with_deg:
                pltpu.sync_copy(deg_sh.at[blk_rows], deg_st)
                pltpu.sync_copy(deg_st, deg_out.at[c, blk_rows])
        for h in wr:
            if h is not None:
                h.wait()

    return pl.kernel(
        body, out_type=out_type, mesh=mesh, scratch_types=scratch,
        compiler_params=pltpu.CompilerParams(use_tc_tiling_on_sc=False),
    )


def kernel(x, edge_index, W_self0, W_neigh0, b0, W_self1, W_neigh1, b1):
    n, d = x.shape
    e = edge_index.shape[1]
    blk = 2560

    np_ = ((n + 1 + NW * 8 - 1) // (NW * 8)) * (NW * 8)
    nch = -(-e // (NW * CHUNK))
    nch = -(-nch // (2 * NBUF)) * (2 * NBUF)
    epad = NW * nch * CHUNK

    pad_idx = n + (jnp.arange(epad - e, dtype=jnp.int32) % (np_ - n))
    src = edge_index[0].astype(jnp.int32)
    dst = edge_index[1].astype(jnp.int32)
    srcs = jnp.concatenate([src, pad_idx]).reshape(NW * nch, CHUNK)
    dsts = jnp.concatenate([dst, pad_idx]).reshape(NW * nch, CHUNK)
    zrow = jnp.zeros((np_, d), jnp.bfloat16)
    zdeg = jnp.zeros((np_, 8), jnp.float32)
    ones8 = jnp.ones((CHUNK, 8), jnp.float32)

    edge_pass0 = _make_edge_pass(np_, d, nch, with_deg=True)
    edge_pass1 = _make_edge_pass(np_, d, nch, with_deg=False)

    s0, z0 = _mm2(x, np_, W_self0, W_neigh0, b0, blk)
    acc0, deg = edge_pass0(z0, srcs, dsts, zrow, zdeg, ones8)
    s1, z1 = _combine_mm(s0, acc0, deg, W_self1, W_neigh1, b1, blk)
    acc1 = edge_pass1(z1, srcs, dsts, zrow, zdeg, ones8)
    if isinstance(acc1, (list, tuple)):
        acc1 = acc1[0]
    return _combine(s1, acc1, deg, n, 2000)
